# baseline (speedup 1.0000x reference)
import functools

import jax
import jax.numpy as jnp
import numpy as np
from jax import lax
from jax.experimental import pallas as pl
from jax.experimental.pallas import tpu as pltpu
from jax.experimental.pallas import tpu_sc as plsc

N = 10000
E = 320000
D = 128

NC = 2
NS = 16
NW = NC * NS
BLK = 112
NBLK = 90
EPW = NBLK * BLK
E_PAD = NW * EPW
N_ACC = 10112
RPS = N_ACC // NS


_MM_ROWS = 2000


def _mm_body(x_ref, wt_ref, b_ref, o_ref):
    o_ref[...] = (
        jnp.dot(
            x_ref[...],
            wt_ref[...],
            preferred_element_type=jnp.float32,
        )
        + b_ref[...]
    )


def _linear(x2d, W, b):
    wt = W.T
    b2 = b.reshape(1, D)
    return pl.pallas_call(
        _mm_body,
        grid=(N // _MM_ROWS,),
        in_specs=[
            pl.BlockSpec((_MM_ROWS, D), lambda i: (i, 0)),
            pl.BlockSpec((D, D), lambda i: (0, 0)),
            pl.BlockSpec((1, D), lambda i: (0, 0)),
        ],
        out_specs=pl.BlockSpec((_MM_ROWS, D), lambda i: (i, 0)),
        out_shape=jax.ShapeDtypeStruct((N, D), jnp.float32),
    )(x2d, wt, b2)


_sc_mesh = plsc.VectorSubcoreMesh(core_axis_name="c", subcore_axis_name="s")


NBUF = 3
NCHUNK = 15
CHB = NBLK // NCHUNK


@functools.partial(
    pl.kernel,
    mesh=_sc_mesh,
    out_type=jax.ShapeDtypeStruct((NC, N_ACC, D), jnp.float32),
    scratch_types=[
        pltpu.VMEM((2, CHB, BLK), jnp.int32),
        pltpu.VMEM((2, CHB, BLK), jnp.int32),
        pltpu.VMEM((NBUF, BLK, D), jnp.float32),
        pltpu.VMEM_SHARED((N_ACC, D), jnp.float32),
    ]
    + [pltpu.SemaphoreType.DMA] * (NBUF + 2),
)
def _sc_spmm(h_hbm, src_hbm, dst_hbm, out_hbm, src_v, dst_v, rows_v, acc, *sems):
    c = lax.axis_index("c")
    s = lax.axis_index("s")
    wid = s * NC + c

    zslot = rows_v.at[NBUF - 1]

    @pl.loop(0, BLK)
    def _(r):
        for cc in range(D // 16):
            rows_v[NBUF - 1, r, pl.ds(cc * 16, 16)] = jnp.zeros((16,), jnp.float32)

    _zcopies = [(t * BLK, BLK) for t in range(RPS // BLK)]
    if RPS % BLK:
        _zcopies.append((RPS - RPS % BLK, RPS % BLK))
    for off, nrows in _zcopies:
        pltpu.async_copy(
            zslot.at[pl.ds(0, nrows)],
            acc.at[pl.ds(s * RPS + off, nrows)],
            sems[NBUF],
        )

    def _idx_prefetch(ch):
        pltpu.async_copy(src_hbm.at[wid, ch], src_v.at[ch % 2], sems[NBUF + 1])
        pltpu.async_copy(dst_hbm.at[wid, ch], dst_v.at[ch % 2], sems[NBUF + 1])

    def _idx_drain():
        pltpu.make_async_copy(src_hbm.at[wid, 0], src_v.at[0], sems[NBUF + 1]).wait()
        pltpu.make_async_copy(dst_hbm.at[wid, 0], dst_v.at[0], sems[NBUF + 1]).wait()

    def _src_row(b):
        return src_v.at[(b // CHB) % 2, b % CHB]

    def _dst_row(b):
        return dst_v.at[(b // CHB) % 2, b % CHB]

    pltpu.sync_copy(src_hbm.at[wid, 0], src_v.at[0])
    pltpu.sync_copy(dst_hbm.at[wid, 0], dst_v.at[0])
    _idx_prefetch(1)

    for k in range(NBUF - 1):
        pltpu.async_copy(h_hbm.at[src_v.at[0, k]], rows_v.at[k], sems[k])

    for off, nrows in _zcopies:
        pltpu.make_async_copy(
            zslot.at[pl.ds(0, nrows)],
            acc.at[pl.ds(s * RPS + off, nrows)],
            sems[NBUF],
        ).wait()
    plsc.subcore_barrier()

    pltpu.async_copy(h_hbm.at[src_v.at[0, NBUF - 1]], rows_v.at[NBUF - 1], sems[NBUF - 1])

    @pl.loop(0, NBLK - NBUF, step=NBUF)
    def _(j):
        rj = j % CHB

        @pl.when(jnp.logical_and(rj == 0, jnp.logical_and(j > 0, j + CHB < NBLK)))
        def _():
            _idx_prefetch(j // CHB + 1)

        @pl.when(rj == CHB - NBUF)
        def _():
            _idx_drain()

        for k in range(NBUF):
            pltpu.make_async_copy(
                h_hbm.at[src_v.at[0, 0]], rows_v.at[k], sems[k]
            ).wait()
            pltpu.sync_copy(rows_v.at[k], acc.at[_dst_row(j + k)], add=True)
            pltpu.async_copy(h_hbm.at[_src_row(j + NBUF + k)], rows_v.at[k], sems[k])

    for k in range(NBUF):
        b = NBLK - NBUF + k
        pltpu.make_async_copy(h_hbm.at[src_v.at[0, 0]], rows_v.at[k], sems[k]).wait()
        pltpu.sync_copy(
            rows_v.at[k], acc.at[dst_v.at[((b // CHB) % 2), b % CHB]], add=True
        )

    plsc.subcore_barrier()

    pltpu.sync_copy(
        acc.at[pl.ds(s * RPS, RPS)], out_hbm.at[c, pl.ds(s * RPS, RPS)]
    )


def _fin_body(p_ref, a_ref, o_ref):
    t = p_ref[0] + p_ref[1]
    o_ref[0] = jnp.where(t >= 0.0, t, a_ref[0, 0] * t)


def _finish(partials, alpha):
    a2 = alpha.reshape(1, 1)
    return pl.pallas_call(
        _fin_body,
        grid=(N // _MM_ROWS,),
        in_specs=[
            pl.BlockSpec((NC, _MM_ROWS, D), lambda i: (0, i, 0)),
            pl.BlockSpec((1, 1), lambda i: (0, 0)),
        ],
        out_specs=pl.BlockSpec((1, _MM_ROWS, D), lambda i: (0, i, 0)),
        out_shape=jax.ShapeDtypeStruct((1, N, D), jnp.float32),
    )(partials, a2)


@jax.jit
def kernel(x, edge_index, W, b, alpha):
    h = _linear(x[0], W, b)

    dst = edge_index[0]
    src = edge_index[1]
    pad = E_PAD - E
    pad_i = np.arange(pad, dtype=np.int32)
    pad_src = jnp.asarray((pad_i * 37) % N, dtype=jnp.int32)
    pad_dst = jnp.asarray(N + (pad_i % (N_ACC - N)), dtype=jnp.int32)
    src_p = jnp.concatenate([src, pad_src]).reshape(NW, NCHUNK, CHB, BLK)
    dst_p = jnp.concatenate([dst, pad_dst]).reshape(NW, NCHUNK, CHB, BLK)

    partials = _sc_spmm(h, src_p, dst_p)

    return _finish(partials, alpha)


# speedup vs baseline: 1.0143x; 1.0143x over previous
import functools

import jax
import jax.numpy as jnp
import numpy as np
from jax import lax
from jax.experimental import pallas as pl
from jax.experimental.pallas import tpu as pltpu
from jax.experimental.pallas import tpu_sc as plsc

N = 10000
E = 320000
D = 128

NC = 2
NS = 16
NW = NC * NS
BLK = 112
NBLK = 90
EPW = NBLK * BLK
E_PAD = NW * EPW
N_ACC = 10112
RPS = N_ACC // NS


_MM_ROWS = 2000


def _mm_body(x_ref, wt_ref, b_ref, o_ref):
    o_ref[...] = (
        jnp.dot(
            x_ref[...],
            wt_ref[...],
            preferred_element_type=jnp.float32,
        )
        + b_ref[...]
    )


def _linear(x2d, W, b):
    wt = W.T
    b2 = b.reshape(1, D)
    return pl.pallas_call(
        _mm_body,
        grid=(N // _MM_ROWS,),
        in_specs=[
            pl.BlockSpec((_MM_ROWS, D), lambda i: (i, 0)),
            pl.BlockSpec((D, D), lambda i: (0, 0)),
            pl.BlockSpec((1, D), lambda i: (0, 0)),
        ],
        out_specs=pl.BlockSpec((_MM_ROWS, D), lambda i: (i, 0)),
        out_shape=jax.ShapeDtypeStruct((N, D), jnp.float32),
    )(x2d, wt, b2)


_sc_mesh = plsc.VectorSubcoreMesh(core_axis_name="c", subcore_axis_name="s")


NBUF = 3
NCHUNK = 15
CHB = NBLK // NCHUNK


@functools.partial(
    pl.kernel,
    mesh=_sc_mesh,
    out_type=jax.ShapeDtypeStruct((NC, N_ACC, D), jnp.float32),
    scratch_types=[
        pltpu.VMEM((2, CHB, BLK), jnp.int32),
        pltpu.VMEM((2, CHB, BLK), jnp.int32),
        pltpu.VMEM((NBUF, BLK, D), jnp.float32),
        pltpu.VMEM_SHARED((N_ACC, D), jnp.float32),
    ]
    + [pltpu.SemaphoreType.DMA] * (NBUF + 2),
)
def _sc_spmm(h_hbm, src_hbm, dst_hbm, out_hbm, src_v, dst_v, rows_v, acc, *sems):
    c = lax.axis_index("c")
    s = lax.axis_index("s")
    wid = s * NC + c

    zslot = rows_v.at[NBUF - 1]

    @pl.loop(0, BLK)
    def _(r):
        for cc in range(D // 16):
            rows_v[NBUF - 1, r, pl.ds(cc * 16, 16)] = jnp.zeros((16,), jnp.float32)

    _zcopies = [(t * BLK, BLK) for t in range(RPS // BLK)]
    if RPS % BLK:
        _zcopies.append((RPS - RPS % BLK, RPS % BLK))
    for off, nrows in _zcopies:
        pltpu.async_copy(
            zslot.at[pl.ds(0, nrows)],
            acc.at[pl.ds(s * RPS + off, nrows)],
            sems[NBUF],
        )

    def _idx_prefetch(ch):
        pltpu.async_copy(src_hbm.at[wid, ch], src_v.at[ch % 2], sems[NBUF + 1])
        pltpu.async_copy(dst_hbm.at[wid, ch], dst_v.at[ch % 2], sems[NBUF + 1])

    def _idx_drain():
        pltpu.make_async_copy(src_hbm.at[wid, 0], src_v.at[0], sems[NBUF + 1]).wait()
        pltpu.make_async_copy(dst_hbm.at[wid, 0], dst_v.at[0], sems[NBUF + 1]).wait()

    def _src_row(b):
        return src_v.at[(b // CHB) % 2, b % CHB]

    def _dst_row(b):
        return dst_v.at[(b // CHB) % 2, b % CHB]

    pltpu.sync_copy(src_hbm.at[wid, 0], src_v.at[0])
    pltpu.sync_copy(dst_hbm.at[wid, 0], dst_v.at[0])
    _idx_prefetch(1)

    for k in range(NBUF - 1):
        pltpu.async_copy(h_hbm.at[src_v.at[0, k]], rows_v.at[k], sems[k])

    for off, nrows in _zcopies:
        pltpu.make_async_copy(
            zslot.at[pl.ds(0, nrows)],
            acc.at[pl.ds(s * RPS + off, nrows)],
            sems[NBUF],
        ).wait()
    plsc.subcore_barrier()

    pltpu.async_copy(h_hbm.at[src_v.at[0, NBUF - 1]], rows_v.at[NBUF - 1], sems[NBUF - 1])

    @pl.loop(0, NBLK - NBUF, step=NBUF)
    def _(j):
        rj = j % CHB

        @pl.when(jnp.logical_and(rj == 0, jnp.logical_and(j > 0, j + CHB < NBLK)))
        def _():
            _idx_prefetch(j // CHB + 1)

        @pl.when(rj == CHB - NBUF)
        def _():
            _idx_drain()

        for k in range(NBUF):
            pltpu.make_async_copy(
                h_hbm.at[src_v.at[0, 0]], rows_v.at[k], sems[k]
            ).wait()
            pltpu.sync_copy(rows_v.at[k], acc.at[_dst_row(j + k)], add=True)
            pltpu.async_copy(h_hbm.at[_src_row(j + NBUF + k)], rows_v.at[k], sems[k])

    for k in range(NBUF):
        b = NBLK - NBUF + k
        pltpu.make_async_copy(h_hbm.at[src_v.at[0, 0]], rows_v.at[k], sems[k]).wait()
        pltpu.sync_copy(
            rows_v.at[k], acc.at[dst_v.at[((b // CHB) % 2), b % CHB]], add=True
        )

    plsc.subcore_barrier()

    pltpu.sync_copy(
        acc.at[pl.ds(s * RPS, RPS)], out_hbm.at[c, pl.ds(s * RPS, RPS)]
    )


_FIN_ROWS = 5000


def _fin_body(p_ref, a_ref, o_ref):
    t = p_ref[0] + p_ref[1]
    o_ref[0] = jnp.where(t >= 0.0, t, a_ref[0, 0] * t)


def _finish(partials, alpha):
    a2 = alpha.reshape(1, 1)
    return pl.pallas_call(
        _fin_body,
        grid=(N // _FIN_ROWS,),
        in_specs=[
            pl.BlockSpec((NC, _FIN_ROWS, D), lambda i: (0, i, 0)),
            pl.BlockSpec((1, 1), lambda i: (0, 0)),
        ],
        out_specs=pl.BlockSpec((1, _FIN_ROWS, D), lambda i: (0, i, 0)),
        out_shape=jax.ShapeDtypeStruct((1, N, D), jnp.float32),
    )(partials, a2)


@jax.jit
def kernel(x, edge_index, W, b, alpha):
    h = _linear(x[0], W, b)

    dst = edge_index[0]
    src = edge_index[1]
    pad = E_PAD - E
    pad_i = np.arange(pad, dtype=np.int32)
    pad_src = jnp.asarray((pad_i * 37) % N, dtype=jnp.int32)
    pad_dst = jnp.asarray(N + (pad_i % (N_ACC - N)), dtype=jnp.int32)
    src_p = jnp.concatenate([src, pad_src]).reshape(NW, NCHUNK, CHB, BLK)
    dst_p = jnp.concatenate([dst, pad_dst]).reshape(NW, NCHUNK, CHB, BLK)

    partials = _sc_spmm(h, src_p, dst_p)

    return _finish(partials, alpha)
